# baseline (speedup 1.0000x reference)
import functools

import jax
import jax.numpy as jnp
from jax import lax
from jax.experimental import pallas as pl
from jax.experimental.pallas import tpu as pltpu

_BF16 = jnp.bfloat16
_EPS = 1e-5
_VMEM = 48 * 1024 * 1024


def _cp():
    return pltpu.CompilerParams(dimension_semantics=("parallel",),
                                vmem_limit_bytes=_VMEM)


def _grid(B):
    return (B,)


def _bmap(B, extra):
    def m(b):
        return (b,) + (0,) * extra
    return m


def _cmap(extra):
    def m(b):
        return (0,) * extra
    return m


def _r8(n):
    return (n + 7) // 8 * 8


def _tap_weights_merged(w):
    c_out, c_in = w.shape[0], w.shape[1]
    wt = jnp.transpose(w, (2, 3, 1, 0)).astype(_BF16)
    taps = [wt[2 * dh:2 * dh + 2, 2 * dw:2 * dw + 2].reshape(4 * c_in, c_out)
            for dh in (0, 1) for dw in (0, 1)]
    return jnp.stack(taps)


def _tap_weights_split(w):
    c_out, c_in = w.shape[0], w.shape[1]
    wt = jnp.transpose(w, (2, 3, 1, 0)).astype(_BF16)
    taps = [wt[2 * dh + p, 2 * dw:2 * dw + 2].reshape(2 * c_in, c_out)
            for p in (0, 1) for dh in (0, 1) for dw in (0, 1)]
    return jnp.stack(taps)


def _stat_mask(mo, mw, ow, c):
    valid = (jnp.arange(mo, dtype=jnp.int32) % mw) < ow
    return jnp.broadcast_to(valid[:, None], (mo, c)).astype(jnp.float32)


def _norm_leaky(h, b_ref, m_ref, nvalid):
    h = h + b_ref[...]
    hm = h * m_ref[...]
    inv_n = 1.0 / nvalid
    mu = jnp.sum(hm, axis=0, keepdims=True) * inv_n
    var = jnp.sum(hm * hm, axis=0, keepdims=True) * inv_n - mu * mu
    h = (h - mu) * lax.rsqrt(var + _EPS)
    return jnp.maximum(h, 0.2 * h)


def _store_parity_merged(hb, oe_ref, oo_ref, oh, mw, ow, c, w2m):
    hf = oh // 2
    wf = ow // 2
    v = hb.reshape(hf, 2, mw, c)[:, :, :ow, :].reshape(hf, 2, wf, 2, c)
    oe_ref[0] = jnp.zeros((hf + 2, w2m, 2 * c), _BF16)
    oo_ref[0] = jnp.zeros((hf + 2, w2m, 2 * c), _BF16)
    oe_ref[0, 1:hf + 1, 1:wf + 1, 0:c] = v[:, 1, :, 1, :].astype(_BF16)
    oe_ref[0, 1:hf + 1, 0:wf, c:2 * c] = v[:, 1, :, 0, :].astype(_BF16)
    oo_ref[0, 0:hf, 1:wf + 1, 0:c] = v[:, 0, :, 1, :].astype(_BF16)
    oo_ref[0, 0:hf, 0:wf, c:2 * c] = v[:, 0, :, 0, :].astype(_BF16)


def _l1_phase_weights(w1):
    c_out, c_in = w1.shape[0], w1.shape[1]
    wt = jnp.transpose(w1, (2, 3, 1, 0)).astype(jnp.float32)
    w4 = jnp.zeros((2, 2, 4, 4, c_in, 2, 2, c_out), jnp.float32)
    for da in range(2):
        for db in range(2):
            for u in range(4):
                for v in range(4):
                    for pp in range(2):
                        for dd in range(2):
                            kh = u + 4 * da - 2 * pp
                            kw = v + 4 * db - 2 * dd
                            if 0 <= kh < 4 and 0 <= kw < 4:
                                w4 = w4.at[da, db, u, v, :, pp, dd, :].set(
                                    wt[kh, kw])
    return w4.reshape(4, 16 * c_in, 4 * c_out).astype(_BF16)


def _l1_body(s_ref, w_ref, b_ref, eo_ref, *, mw, mo, oh, ow, c, w2m):
    shifts = (0, 1, mw, mw + 1)
    h = jnp.dot(s_ref[0, pl.ds(0, mo), :], w_ref[0],
                preferred_element_type=jnp.float32)
    for t in range(1, 4):
        h = h + jnp.dot(s_ref[0, pl.ds(shifts[t], mo), :], w_ref[t],
                        preferred_element_type=jnp.float32)
    h = h + b_ref[...]
    h = jnp.maximum(h, 0.2 * h)
    hf = oh // 2
    wf = ow // 2
    h3 = h.reshape(hf, mw, 4 * c)[:, :wf, :]
    eo_ref[0] = jnp.zeros((hf + 2, w2m, 4 * c), _BF16)
    eo_ref[0, 1:hf + 1, 1:wf + 1, 0:c] = h3[:, :, 3 * c:4 * c].astype(_BF16)
    eo_ref[0, 1:hf + 1, 0:wf, c:2 * c] = h3[:, :, 2 * c:3 * c].astype(_BF16)
    eo_ref[0, 0:hf, 1:wf + 1, 2 * c:3 * c] = h3[:, :, c:2 * c].astype(_BF16)
    eo_ref[0, 0:hf, 0:wf, 3 * c:4 * c] = h3[:, :, 0:c].astype(_BF16)


def _layer1(x, w1, b1):
    B, H, W, C = x.shape
    oh, ow = H // 2, W // 2
    hf, wf = oh // 2, ow // 2
    c_out = w1.shape[0]
    xp = jnp.pad(x, ((0, 0), (1, 3), (1, 3), (0, 0)))
    s4 = xp.reshape(B, hf + 1, 4, wf + 1, 4, C)
    s4 = s4.transpose(0, 1, 3, 2, 4, 5).reshape(B, hf + 1, wf + 1, 16 * C)
    mw = _r8(wf + 1)
    s4 = jnp.pad(s4, ((0, 0), (0, 1), (0, mw - (wf + 1)), (0, 0)))
    s4 = s4.reshape(B, (hf + 2) * mw, 16 * C)
    mo = hf * mw
    w_taps = _l1_phase_weights(w1)
    bias = jnp.tile(b1.astype(jnp.float32), 4).reshape(1, 4 * c_out)
    rr = hf + 2
    w2m = _r8(wf + 1)
    body = functools.partial(_l1_body, mw=mw, mo=mo, oh=oh, ow=ow, c=c_out,
                             w2m=w2m)
    eo = pl.pallas_call(
        body,
        out_shape=jax.ShapeDtypeStruct((B, rr, w2m, 4 * c_out), _BF16),
        grid=_grid(B),
        in_specs=[
            pl.BlockSpec((1, (hf + 2) * mw, 16 * C), _bmap(B, 2)),
            pl.BlockSpec((4, 16 * C, 4 * c_out), _cmap(3)),
            pl.BlockSpec((1, 4 * c_out), _cmap(2)),
        ],
        out_specs=pl.BlockSpec((1, rr, w2m, 4 * c_out), _bmap(B, 3)),
        compiler_params=_cp(),
    )(s4, w_taps, bias)
    return eo.reshape(B, rr * w2m, 4 * c_out), w2m


def _l2_body(eo_ref, w_ref, b_ref, m_ref, oe_ref, oo_ref, *, mw, mo, oh, ow,
             nvalid, c, w2m):
    shifts = (0, 1, mw, mw + 1)
    h = jnp.dot(eo_ref[0, pl.ds(0, mo), :], w_ref[0],
                preferred_element_type=jnp.float32)
    for t in range(1, 4):
        h = h + jnp.dot(eo_ref[0, pl.ds(shifts[t], mo), :], w_ref[t],
                        preferred_element_type=jnp.float32)
    h = _norm_leaky(h, b_ref, m_ref, nvalid)
    _store_parity_merged(h, oe_ref, oo_ref, oh, mw, ow, c, w2m)


def _layer2(eo_flat, w, b, oh, ow, mw):
    B, L, k4 = eo_flat.shape
    c_out = w.shape[0]
    mo = oh * mw
    w_taps = _tap_weights_merged(w)
    mask = _stat_mask(mo, mw, ow, c_out)
    rr = oh // 2 + 2
    w2m = _r8(ow // 2 + 1)
    body = functools.partial(_l2_body, mw=mw, mo=mo, oh=oh, ow=ow,
                             nvalid=oh * ow, c=c_out, w2m=w2m)
    osd = jax.ShapeDtypeStruct((B, rr, w2m, 2 * c_out), _BF16)
    obs = pl.BlockSpec((1, rr, w2m, 2 * c_out), _bmap(B, 3))
    e, o = pl.pallas_call(
        body,
        out_shape=(osd, osd),
        grid=_grid(B),
        in_specs=[
            pl.BlockSpec((1, L, k4), _bmap(B, 2)),
            pl.BlockSpec((4, k4, c_out), _cmap(3)),
            pl.BlockSpec((1, c_out), _cmap(2)),
            pl.BlockSpec((mo, c_out), _cmap(2)),
        ],
        out_specs=(obs, obs),
        compiler_params=_cp(),
    )(eo_flat, w_taps, b.reshape(1, c_out).astype(jnp.float32), mask)
    return (e.reshape(B, rr * w2m, 2 * c_out),
            o.reshape(B, rr * w2m, 2 * c_out), w2m)


def _l3_body(e_ref, o_ref, w_ref, b_ref, m_ref, oe_ref, oo_ref, *, mw, mo,
             oh, ow, nvalid, c, w2m):
    h = None
    for i, (dh, dw) in enumerate(((0, 0), (0, 1), (1, 0), (1, 1))):
        s = dh * mw + dw
        d = jnp.dot(e_ref[0, pl.ds(s, mo), :], w_ref[i],
                    preferred_element_type=jnp.float32)
        h = d if h is None else h + d
        h = h + jnp.dot(o_ref[0, pl.ds(s, mo), :], w_ref[4 + i],
                        preferred_element_type=jnp.float32)
    h = _norm_leaky(h, b_ref, m_ref, nvalid)
    _store_parity_merged(h, oe_ref, oo_ref, oh, mw, ow, c, w2m)


def _layer3(e_flat, o_flat, w, b, oh, ow, mw):
    B, L, k2 = e_flat.shape
    c_out = w.shape[0]
    mo = oh * mw
    w_taps = _tap_weights_split(w)
    mask = _stat_mask(mo, mw, ow, c_out)
    rr = oh // 2 + 2
    w2m = _r8(ow // 2 + 1)
    body = functools.partial(_l3_body, mw=mw, mo=mo, oh=oh, ow=ow,
                             nvalid=oh * ow, c=c_out, w2m=w2m)
    osd = jax.ShapeDtypeStruct((B, rr, w2m, 2 * c_out), _BF16)
    obs = pl.BlockSpec((1, rr, w2m, 2 * c_out), _bmap(B, 3))
    ibs = pl.BlockSpec((1, L, k2), _bmap(B, 2))
    e, o = pl.pallas_call(
        body,
        out_shape=(osd, osd),
        grid=_grid(B),
        in_specs=[
            ibs, ibs,
            pl.BlockSpec((8, k2, c_out), _cmap(3)),
            pl.BlockSpec((1, c_out), _cmap(2)),
            pl.BlockSpec((mo, c_out), _cmap(2)),
        ],
        out_specs=(obs, obs),
        compiler_params=_cp(),
    )(e_flat, o_flat, w_taps, b.reshape(1, c_out).astype(jnp.float32), mask)
    return (e.reshape(B, rr * w2m, 2 * c_out),
            o.reshape(B, rr * w2m, 2 * c_out), w2m)


def _l4_body(e_ref, o_ref, w_ref, b_ref, m_ref, o5_ref, *, mw, mo, oh, ow,
             nvalid, c, wp):
    h = None
    for i, (dh, dw) in enumerate(((0, 0), (0, 1), (1, 0), (1, 1))):
        s = dh * mw + dw
        d = jnp.dot(e_ref[0, pl.ds(s, mo), :], w_ref[i],
                    preferred_element_type=jnp.float32)
        h = d if h is None else h + d
        h = h + jnp.dot(o_ref[0, pl.ds(s, mo), :], w_ref[4 + i],
                        preferred_element_type=jnp.float32)
    h = _norm_leaky(h, b_ref, m_ref, nvalid)
    hb = h.astype(_BF16).reshape(oh, mw, c)[:, :ow, :]
    o5_ref[0] = jnp.zeros((oh + 4, wp, c), _BF16)
    o5_ref[0, 2:oh + 2, 2:ow + 2, :] = hb


def _layer4(e_flat, o_flat, w, b, oh, ow, mw):
    B, L, k2 = e_flat.shape
    c_out = w.shape[0]
    mo = oh * mw
    w_taps = _tap_weights_split(w)
    mask = _stat_mask(mo, mw, ow, c_out)
    wp = _r8(ow + 3)
    body = functools.partial(_l4_body, mw=mw, mo=mo, oh=oh, ow=ow,
                             nvalid=oh * ow, c=c_out, wp=wp)
    osd = jax.ShapeDtypeStruct((B, oh + 4, wp, c_out), _BF16)
    obs = pl.BlockSpec((1, oh + 4, wp, c_out), _bmap(B, 3))
    ibs = pl.BlockSpec((1, L, k2), _bmap(B, 2))
    out = pl.pallas_call(
        body,
        out_shape=osd,
        grid=_grid(B),
        in_specs=[
            ibs, ibs,
            pl.BlockSpec((8, k2, c_out), _cmap(3)),
            pl.BlockSpec((1, c_out), _cmap(2)),
            pl.BlockSpec((mo, c_out), _cmap(2)),
        ],
        out_specs=obs,
        compiler_params=_cp(),
    )(e_flat, o_flat, w_taps, b.reshape(1, c_out).astype(jnp.float32), mask)
    return out.reshape(B, (oh + 4) * wp, c_out), wp


def _l5_body(x_ref, w_ref, o_ref, *, wp, mo):
    h = None
    for kh in range(4):
        for kw in range(4):
            t = kh * 4 + kw
            d = jnp.dot(x_ref[0, pl.ds(kh * wp + kw, mo), :], w_ref[t],
                        preferred_element_type=jnp.float32)
            h = d if h is None else h + d
    o_ref[0] = jax.nn.sigmoid(h)


def _layer5(flat, w5, hh, ww, wp):
    B, L, C = flat.shape
    mo = hh * wp
    wt = jnp.transpose(w5, (2, 3, 1, 0)).astype(_BF16)
    w_taps = jnp.stack([jnp.pad(wt[kh, kw], ((0, 0), (0, 7)))
                        for kh in range(4) for kw in range(4)])
    body = functools.partial(_l5_body, wp=wp, mo=mo)
    out = pl.pallas_call(
        body,
        out_shape=jax.ShapeDtypeStruct((B, mo, 8), jnp.float32),
        grid=_grid(B),
        in_specs=[
            pl.BlockSpec((1, L, C), _bmap(B, 2)),
            pl.BlockSpec((16, C, 8), _cmap(3)),
        ],
        out_specs=pl.BlockSpec((1, mo, 8), _bmap(B, 2)),
        compiler_params=_cp(),
    )(flat, w_taps)
    return out[:, :, 0].reshape(B, hh, wp)[:, :, :ww].reshape(B, 1, hh, ww)


def kernel(w1, b1, w2, b2, w3, b3, w4, b4, w5, img_A, img_B):
    B, _, H, W = img_A.shape
    oh2, ow2 = H // 4, W // 4
    oh3, ow3 = H // 8, W // 8
    oh4, ow4 = H // 16, W // 16
    x = jnp.concatenate([img_A, img_B], axis=1).astype(_BF16)
    x = jnp.transpose(x, (0, 2, 3, 1))

    eo2, mw2 = _layer1(x, w1, b1)
    e2, o2, mw3 = _layer2(eo2, w2, b2, oh2, ow2, mw2)
    e3, o3, mw4 = _layer3(e2, o2, w3, b3, oh3, ow3, mw3)
    x5, wp = _layer4(e3, o3, w4, b4, oh4, ow4, mw4)
    return _layer5(x5, w5, oh4, ow4, wp)
